# baseline (speedup 1.0000x reference)
import jax
import jax.numpy as jnp
from jax import lax
from jax.experimental import pallas as pl
from jax.experimental.pallas import tpu as pltpu
from jax.experimental.pallas import tpu_sc as plsc

N = 50000
E = 800000
IN_DIM = 16
OUT_DIM = 16
EDGE_DIM = 16

NC = 2
NS = 16
NW = NC * NS

ICHUNK = 128
IROWS = 6272
E_PAD = IROWS * ICHUNK
NSUB = 3136
N_PAD = NS * NSUB
NR = N_PAD // 16
NR_REAL = N // 16

WROWS = IROWS // NW
BROWS = 7
NBLK = WROWS // BROWS

BE = 4096

_sc_mesh = plsc.VectorSubcoreMesh(core_axis_name="c", subcore_axis_name="s")
_sc_params = pltpu.CompilerParams(use_tc_tiling_on_sc=False,
                                  needs_layout_passes=False)


def _gather_body(node_hbm, src_hbm, dst_hbm, ones_hbm, zeros_hbm,
                 xjt_hbm, cnts_hbm,
                 idx_v, rows_v, colall_v, ones_v, node_v, cacc, sem):
    cid = lax.axis_index("c")
    sid = lax.axis_index("s")
    wid = sid * NC + cid
    base = wid * WROWS
    nrow0 = sid * NSUB

    pltpu.sync_copy(zeros_hbm, node_v)
    pltpu.sync_copy(node_v, cacc.at[pl.ds(nrow0, NSUB)])
    pltpu.sync_copy(ones_hbm, ones_v)
    plsc.subcore_barrier()

    lane = lax.iota(jnp.int32, 16)

    def blk(j, _):
        row0 = base + j * BROWS
        pltpu.sync_copy(src_hbm.at[pl.ds(row0, BROWS)], idx_v)
        copies = [
            pltpu.async_copy(node_hbm.at[idx_v.at[jj]],
                             rows_v.at[pl.ds(jj * ICHUNK, ICHUNK)], sem)
            for jj in range(BROWS)
        ]
        for cp in copies:
            cp.wait()
        for tr in range(2):
            for r in range(8):
                feat = jnp.full((16,), tr * 8 + r, jnp.int32)
                for g in range(BROWS * ICHUNK // 16):
                    vals = plsc.load_gather(rows_v, [g * 16 + lane, feat])
                    colall_v[r * BROWS + g // 8,
                             pl.ds((g % 8) * 16, 16)] = vals
            for r in range(8):
                pltpu.sync_copy(colall_v.at[pl.ds(r * BROWS, BROWS)],
                                xjt_hbm.at[tr, pl.ds(row0, BROWS), r])
        pltpu.sync_copy(dst_hbm.at[pl.ds(row0, BROWS)], idx_v)
        for jj in range(BROWS):
            pltpu.sync_copy(ones_v, cacc.at[idx_v.at[jj]], add=True)
        return _

    lax.fori_loop(0, NBLK, blk, None)
    plsc.subcore_barrier()

    pltpu.sync_copy(cacc.at[pl.ds(nrow0, NSUB)], node_v)
    pltpu.sync_copy(node_v, cnts_hbm.at[cid, pl.ds(nrow0, NSUB)])


_gather = pl.kernel(
    _gather_body,
    out_type=(
        jax.ShapeDtypeStruct((2, IROWS, 8, 128), jnp.float32),
        jax.ShapeDtypeStruct((NC, N_PAD, OUT_DIM), jnp.float32),
    ),
    mesh=_sc_mesh,
    compiler_params=_sc_params,
    scratch_types=[
        pltpu.VMEM((BROWS, ICHUNK), jnp.int32),
        pltpu.VMEM((BROWS * ICHUNK, IN_DIM), jnp.float32),
        pltpu.VMEM((8 * BROWS, 128), jnp.float32),
        pltpu.VMEM((ICHUNK, OUT_DIM), jnp.float32),
        pltpu.VMEM((NSUB, OUT_DIM), jnp.float32),
        pltpu.VMEM_SHARED((N_PAD, OUT_DIM), jnp.float32),
        pltpu.SemaphoreType.DMA,
    ],
)


def _scatter_body(msgs_hbm, dst_hbm, zeros_hbm, sums_hbm,
                  idx_v, mall_v, msg_v, node_v, acc):
    cid = lax.axis_index("c")
    sid = lax.axis_index("s")
    wid = sid * NC + cid
    base = wid * WROWS
    nrow0 = sid * NSUB

    pltpu.sync_copy(zeros_hbm, node_v)
    pltpu.sync_copy(node_v, acc.at[pl.ds(nrow0, NSUB)])
    plsc.subcore_barrier()

    lane = lax.iota(jnp.int32, 16)
    feat_row = lane * BROWS

    def blk(j, _):
        row0 = base + j * BROWS
        pltpu.sync_copy(dst_hbm.at[pl.ds(row0, BROWS)], idx_v)
        for tr in range(2):
            for r in range(8):
                f = tr * 8 + r
                pltpu.sync_copy(msgs_hbm.at[tr, pl.ds(row0, BROWS), r],
                                mall_v.at[pl.ds(f * BROWS, BROWS)])
        for e in range(BROWS * ICHUNK):
            vals = plsc.load_gather(
                mall_v, [feat_row + (e // 128),
                         jnp.full((16,), e % 128, jnp.int32)])
            msg_v[e, :] = vals
        for jj in range(BROWS):
            pltpu.sync_copy(msg_v.at[pl.ds(jj * ICHUNK, ICHUNK)],
                            acc.at[idx_v.at[jj]], add=True)
        return _

    lax.fori_loop(0, NBLK, blk, None)
    plsc.subcore_barrier()

    pltpu.sync_copy(acc.at[pl.ds(nrow0, NSUB)], node_v)
    pltpu.sync_copy(node_v, sums_hbm.at[cid, pl.ds(nrow0, NSUB)])


_scatter = pl.kernel(
    _scatter_body,
    out_type=jax.ShapeDtypeStruct((NC, N_PAD, OUT_DIM), jnp.float32),
    mesh=_sc_mesh,
    compiler_params=_sc_params,
    scratch_types=[
        pltpu.VMEM((BROWS, ICHUNK), jnp.int32),
        pltpu.VMEM((16 * BROWS, 128), jnp.float32),
        pltpu.VMEM((BROWS * ICHUNK, OUT_DIM), jnp.float32),
        pltpu.VMEM((NSUB, OUT_DIM), jnp.float32),
        pltpu.VMEM_SHARED((N_PAD, OUT_DIM), jnp.float32),
    ],
)


_CONTRACT_LAST = (((1,), (1,)), ((), ()))
NBCH = BE // 128


def _dense_body(ef_ref, xjt_ref, wt_ref, bt_ref, out_ref):
    t = jnp.tanh(
        lax.dot_general(wt_ref[...], ef_ref[...], _CONTRACT_LAST,
                        preferred_element_type=jnp.float32) + bt_ref[...])
    for l in range(NBCH):
        tl = t[:, l * 128:(l + 1) * 128]
        acc = None
        for i in range(IN_DIM):
            xr = xjt_ref[i // 8, l, i % 8]
            xb = jnp.broadcast_to(xr[None, :], (OUT_DIM, 128))
            term = xb * tl[i * OUT_DIM:(i + 1) * OUT_DIM, :]
            acc = term if acc is None else acc + term
        out_ref[:, l, :, :] = acc.reshape(2, 8, 128)


def _dense(ef, xjt4, wt, bt):
    grid = (E_PAD // BE,)
    return pl.pallas_call(
        _dense_body,
        grid=grid,
        in_specs=[
            pl.BlockSpec((BE, EDGE_DIM), lambda i: (i, 0)),
            pl.BlockSpec((2, NBCH, 8, 128), lambda i: (0, i, 0, 0)),
            pl.BlockSpec((IN_DIM * OUT_DIM, EDGE_DIM), lambda i: (0, 0)),
            pl.BlockSpec((IN_DIM * OUT_DIM, 1), lambda i: (0, 0)),
        ],
        out_specs=pl.BlockSpec((2, NBCH, 8, 128), lambda i: (0, i, 0, 0)),
        out_shape=jax.ShapeDtypeStruct((2, IROWS, 8, 128), jnp.float32),
    )(ef, xjt4, wt, bt)


def _finish_body(sums_ref, cnts_ref, node_ref, wbig_ref, bbig_ref,
                 gbig_ref, betab_ref, fold_ref, unfold_ref, out_ref):
    s = sums_ref[0] + sums_ref[1]
    c = cnts_ref[0] + cnts_ref[1]
    aggr = s / jnp.maximum(c, 1.0)
    root = jnp.dot(node_ref[...], wbig_ref[...],
                   preferred_element_type=jnp.float32,
                   precision=lax.Precision.HIGHEST)
    pre = aggr + root + bbig_ref[...]
    rid = lax.broadcasted_iota(jnp.int32, (NR, IN_DIM * OUT_DIM), 0)
    pre = jnp.where(rid < NR_REAL, pre, 0.0)
    colsum = jnp.sum(pre, axis=0, keepdims=True)
    colsq = jnp.sum(pre * pre, axis=0, keepdims=True)
    tot = jnp.dot(colsum, fold_ref[...], preferred_element_type=jnp.float32,
                  precision=lax.Precision.HIGHEST)
    totsq = jnp.dot(colsq, fold_ref[...], preferred_element_type=jnp.float32,
                    precision=lax.Precision.HIGHEST)
    mean16 = tot / float(N)
    var16 = totsq / float(N) - mean16 * mean16
    mean_b = jnp.dot(mean16, unfold_ref[...],
                     preferred_element_type=jnp.float32,
                     precision=lax.Precision.HIGHEST)
    var_b = jnp.dot(var16, unfold_ref[...],
                    preferred_element_type=jnp.float32,
                    precision=lax.Precision.HIGHEST)
    y = (pre - mean_b) * lax.rsqrt(var_b + 1e-5) * gbig_ref[...] \
        + betab_ref[...]
    out_ref[...] = jnp.where(y >= 0.0, y, 0.01 * y)


def _finish(sums_r, cnts_r, node_r, wbig, bbig, gbig, betab, fold, unfold):
    return pl.pallas_call(
        _finish_body,
        out_shape=jax.ShapeDtypeStruct((NR, IN_DIM * OUT_DIM), jnp.float32),
    )(sums_r, cnts_r, node_r, wbig, bbig, gbig, betab, fold, unfold)


def kernel(node_feat, edge_feat, edge_index, batch_index,
           W_edge, b_edge, W_root, b_root, bn_gamma, bn_beta):
    del batch_index
    epad = E_PAD - E
    src = jnp.concatenate(
        [edge_index[0], jnp.zeros((epad,), edge_index.dtype)]
    ).astype(jnp.int32).reshape(IROWS, ICHUNK)
    dst = jnp.concatenate(
        [edge_index[1], jnp.full((epad,), N, edge_index.dtype)]
    ).astype(jnp.int32).reshape(IROWS, ICHUNK)

    node_p = jnp.pad(node_feat, ((0, N_PAD - N), (0, 0)))
    ones_rows = jnp.ones((ICHUNK, OUT_DIM), jnp.float32)
    zeros_rows = jnp.zeros((NSUB, OUT_DIM), jnp.float32)

    xjt4, cnts = _gather(node_p, src, dst, ones_rows, zeros_rows)

    ef_p = jnp.pad(edge_feat, ((0, epad), (0, 0)))
    wt = W_edge.T
    bt = b_edge.reshape(IN_DIM * OUT_DIM, 1)
    msgst4 = _dense(ef_p, xjt4, wt, bt)

    sums = _scatter(msgst4, dst, zeros_rows)

    eye = jnp.eye(IN_DIM, dtype=jnp.float32)
    wbig = jnp.kron(eye, W_root)
    fold = jnp.kron(jnp.ones((IN_DIM, 1), jnp.float32), eye)
    unfold = fold.T
    bbig = jnp.tile(b_root, IN_DIM).reshape(1, IN_DIM * OUT_DIM)
    gbig = jnp.tile(bn_gamma, IN_DIM).reshape(1, IN_DIM * OUT_DIM)
    betab = jnp.tile(bn_beta, IN_DIM).reshape(1, IN_DIM * OUT_DIM)

    lanes = IN_DIM * OUT_DIM
    out_r = _finish(
        sums.reshape(NC, NR, lanes), cnts.reshape(NC, NR, lanes),
        node_p.reshape(NR, lanes), wbig, bbig, gbig, betab, fold, unfold)
    return out_r.reshape(N_PAD, OUT_DIM)[:N]


# speedup vs baseline: 1.2278x; 1.2278x over previous
import jax
import jax.numpy as jnp
from jax import lax
from jax.experimental import pallas as pl
from jax.experimental.pallas import tpu as pltpu
from jax.experimental.pallas import tpu_sc as plsc

N = 50000
E = 800000
IN_DIM = 16
OUT_DIM = 16
EDGE_DIM = 16

NC = 2
NS = 16
NW = NC * NS

ICHUNK = 128
IROWS = 6272
E_PAD = IROWS * ICHUNK
NSUB = 3136
N_PAD = NS * NSUB
NR = N_PAD // 16
NR_REAL = N // 16

WROWS = IROWS // NW
BROWS = 7
NBLK = WROWS // BROWS

BE = 4096

_sc_mesh = plsc.VectorSubcoreMesh(core_axis_name="c", subcore_axis_name="s")
_sc_params = pltpu.CompilerParams(use_tc_tiling_on_sc=False,
                                  needs_layout_passes=False)


def _gather_body(node_hbm, src_hbm, xjt_hbm,
                 idx_v, rows_v, colall_v, sem):
    cid = lax.axis_index("c")
    sid = lax.axis_index("s")
    wid = sid * NC + cid
    base = wid * WROWS

    lane_b = lax.iota(jnp.int32, 16) * BROWS

    def blk(j, _):
        row0 = base + j * BROWS
        pltpu.sync_copy(src_hbm.at[pl.ds(row0, BROWS)], idx_v)
        copies = [
            pltpu.async_copy(node_hbm.at[idx_v.at[jj]],
                             rows_v.at[pl.ds(jj * ICHUNK, ICHUNK)], sem)
            for jj in range(BROWS)
        ]
        for cp in copies:
            cp.wait()
        for e in range(BROWS * ICHUNK):
            vals = rows_v[e, :]
            plsc.store_scatter(
                colall_v,
                [lane_b + (e // 128), jnp.full((16,), e % 128, jnp.int32)],
                vals)
        for tr in range(2):
            for r in range(8):
                f = tr * 8 + r
                pltpu.sync_copy(colall_v.at[pl.ds(f * BROWS, BROWS),
                                            pl.ds(0, 128)],
                                xjt_hbm.at[tr, pl.ds(row0, BROWS), r])
        return _

    lax.fori_loop(0, NBLK, blk, None)


_gather = pl.kernel(
    _gather_body,
    out_type=jax.ShapeDtypeStruct((2, IROWS, 8, 128), jnp.float32),
    mesh=_sc_mesh,
    compiler_params=_sc_params,
    scratch_types=[
        pltpu.VMEM((BROWS, ICHUNK), jnp.int32),
        pltpu.VMEM((BROWS * ICHUNK, IN_DIM), jnp.float32),
        pltpu.VMEM((16 * BROWS, 129), jnp.float32),
        pltpu.SemaphoreType.DMA,
    ],
)


def _scatter_ones_body(dst_hbm, ones_hbm, zeros_hbm, cnts_hbm,
                       idx_v, ones_v, node_v, acc):
    cid = lax.axis_index("c")
    sid = lax.axis_index("s")
    wid = sid * NC + cid
    base = wid * WROWS
    nrow0 = sid * NSUB

    pltpu.sync_copy(zeros_hbm, node_v)
    pltpu.sync_copy(node_v, acc.at[pl.ds(nrow0, NSUB)])
    pltpu.sync_copy(ones_hbm, ones_v)
    plsc.subcore_barrier()

    def blk(j, _):
        row0 = base + j * BROWS
        pltpu.sync_copy(dst_hbm.at[pl.ds(row0, BROWS)], idx_v)
        for jj in range(BROWS):
            pltpu.sync_copy(ones_v, acc.at[idx_v.at[jj]], add=True)
        return _

    lax.fori_loop(0, NBLK, blk, None)
    plsc.subcore_barrier()

    pltpu.sync_copy(acc.at[pl.ds(nrow0, NSUB)], node_v)
    pltpu.sync_copy(node_v, cnts_hbm.at[cid, pl.ds(nrow0, NSUB)])


_scatter_ones = pl.kernel(
    _scatter_ones_body,
    out_type=jax.ShapeDtypeStruct((NC, N_PAD, OUT_DIM), jnp.float32),
    mesh=_sc_mesh,
    compiler_params=_sc_params,
    scratch_types=[
        pltpu.VMEM((BROWS, ICHUNK), jnp.int32),
        pltpu.VMEM((ICHUNK, OUT_DIM), jnp.float32),
        pltpu.VMEM((NSUB, OUT_DIM), jnp.float32),
        pltpu.VMEM_SHARED((N_PAD, OUT_DIM), jnp.float32),
    ],
)


def _scatter_body(msgs_hbm, dst_hbm, zeros_hbm, sums_hbm,
                  idx_v, mall_v, msg_v, node_v, acc):
    cid = lax.axis_index("c")
    sid = lax.axis_index("s")
    wid = sid * NC + cid
    base = wid * WROWS
    nrow0 = sid * NSUB

    pltpu.sync_copy(zeros_hbm, node_v)
    pltpu.sync_copy(node_v, acc.at[pl.ds(nrow0, NSUB)])
    plsc.subcore_barrier()

    lane = lax.iota(jnp.int32, 16)
    feat_row = lane * BROWS

    def blk(j, _):
        row0 = base + j * BROWS
        pltpu.sync_copy(dst_hbm.at[pl.ds(row0, BROWS)], idx_v)
        for tr in range(2):
            for r in range(8):
                f = tr * 8 + r
                pltpu.sync_copy(msgs_hbm.at[tr, pl.ds(row0, BROWS), r],
                                mall_v.at[pl.ds(f * BROWS, BROWS),
                                          pl.ds(0, 128)])
        for e in range(BROWS * ICHUNK):
            vals = plsc.load_gather(
                mall_v, [feat_row + (e // 128),
                         jnp.full((16,), e % 128, jnp.int32)])
            msg_v[e, :] = vals
        for jj in range(BROWS):
            pltpu.sync_copy(msg_v.at[pl.ds(jj * ICHUNK, ICHUNK)],
                            acc.at[idx_v.at[jj]], add=True)
        return _

    lax.fori_loop(0, NBLK, blk, None)
    plsc.subcore_barrier()

    pltpu.sync_copy(acc.at[pl.ds(nrow0, NSUB)], node_v)
    pltpu.sync_copy(node_v, sums_hbm.at[cid, pl.ds(nrow0, NSUB)])


_scatter = pl.kernel(
    _scatter_body,
    out_type=jax.ShapeDtypeStruct((NC, N_PAD, OUT_DIM), jnp.float32),
    mesh=_sc_mesh,
    compiler_params=_sc_params,
    scratch_types=[
        pltpu.VMEM((BROWS, ICHUNK), jnp.int32),
        pltpu.VMEM((16 * BROWS, 129), jnp.float32),
        pltpu.VMEM((BROWS * ICHUNK, OUT_DIM), jnp.float32),
        pltpu.VMEM((NSUB, OUT_DIM), jnp.float32),
        pltpu.VMEM_SHARED((N_PAD, OUT_DIM), jnp.float32),
    ],
)


_CONTRACT_LAST = (((1,), (1,)), ((), ()))
NBCH = BE // 128


def _dense_body(ef_ref, xjt_ref, wt_ref, bt_ref, out_ref):
    t = jnp.tanh(
        lax.dot_general(wt_ref[...], ef_ref[...], _CONTRACT_LAST,
                        preferred_element_type=jnp.float32) + bt_ref[...])
    for l in range(NBCH):
        tl = t[:, l * 128:(l + 1) * 128]
        acc = None
        for i in range(IN_DIM):
            xr = xjt_ref[i // 8, l, i % 8]
            xb = jnp.broadcast_to(xr[None, :], (OUT_DIM, 128))
            term = xb * tl[i * OUT_DIM:(i + 1) * OUT_DIM, :]
            acc = term if acc is None else acc + term
        out_ref[:, l, :, :] = acc.reshape(2, 8, 128)


def _dense(ef, xjt4, wt, bt):
    grid = (E_PAD // BE,)
    return pl.pallas_call(
        _dense_body,
        grid=grid,
        in_specs=[
            pl.BlockSpec((BE, EDGE_DIM), lambda i: (i, 0)),
            pl.BlockSpec((2, NBCH, 8, 128), lambda i: (0, i, 0, 0)),
            pl.BlockSpec((IN_DIM * OUT_DIM, EDGE_DIM), lambda i: (0, 0)),
            pl.BlockSpec((IN_DIM * OUT_DIM, 1), lambda i: (0, 0)),
        ],
        out_specs=pl.BlockSpec((2, NBCH, 8, 128), lambda i: (0, i, 0, 0)),
        out_shape=jax.ShapeDtypeStruct((2, IROWS, 8, 128), jnp.float32),
    )(ef, xjt4, wt, bt)


def _finish_body(sums_ref, cnts_ref, node_ref, wbig_ref, bbig_ref,
                 gbig_ref, betab_ref, fold_ref, unfold_ref, out_ref):
    s = sums_ref[0] + sums_ref[1]
    c = cnts_ref[0] + cnts_ref[1]
    aggr = s / jnp.maximum(c, 1.0)
    root = jnp.dot(node_ref[...], wbig_ref[...],
                   preferred_element_type=jnp.float32,
                   precision=lax.Precision.HIGHEST)
    pre = aggr + root + bbig_ref[...]
    rid = lax.broadcasted_iota(jnp.int32, (NR, IN_DIM * OUT_DIM), 0)
    pre = jnp.where(rid < NR_REAL, pre, 0.0)
    colsum = jnp.sum(pre, axis=0, keepdims=True)
    colsq = jnp.sum(pre * pre, axis=0, keepdims=True)
    tot = jnp.dot(colsum, fold_ref[...], preferred_element_type=jnp.float32,
                  precision=lax.Precision.HIGHEST)
    totsq = jnp.dot(colsq, fold_ref[...], preferred_element_type=jnp.float32,
                    precision=lax.Precision.HIGHEST)
    mean16 = tot / float(N)
    var16 = totsq / float(N) - mean16 * mean16
    mean_b = jnp.dot(mean16, unfold_ref[...],
                     preferred_element_type=jnp.float32,
                     precision=lax.Precision.HIGHEST)
    var_b = jnp.dot(var16, unfold_ref[...],
                    preferred_element_type=jnp.float32,
                    precision=lax.Precision.HIGHEST)
    y = (pre - mean_b) * lax.rsqrt(var_b + 1e-5) * gbig_ref[...] \
        + betab_ref[...]
    out_ref[...] = jnp.where(y >= 0.0, y, 0.01 * y)


def _finish(sums_r, cnts_r, node_r, wbig, bbig, gbig, betab, fold, unfold):
    return pl.pallas_call(
        _finish_body,
        out_shape=jax.ShapeDtypeStruct((NR, IN_DIM * OUT_DIM), jnp.float32),
    )(sums_r, cnts_r, node_r, wbig, bbig, gbig, betab, fold, unfold)


def kernel(node_feat, edge_feat, edge_index, batch_index,
           W_edge, b_edge, W_root, b_root, bn_gamma, bn_beta):
    del batch_index
    epad = E_PAD - E
    src = jnp.concatenate(
        [edge_index[0], jnp.zeros((epad,), edge_index.dtype)]
    ).astype(jnp.int32).reshape(IROWS, ICHUNK)
    dst = jnp.concatenate(
        [edge_index[1], jnp.full((epad,), N, edge_index.dtype)]
    ).astype(jnp.int32).reshape(IROWS, ICHUNK)

    node_p = jnp.pad(node_feat, ((0, N_PAD - N), (0, 0)))
    ones_rows = jnp.ones((ICHUNK, OUT_DIM), jnp.float32)
    zeros_rows = jnp.zeros((NSUB, OUT_DIM), jnp.float32)

    xjt4 = _gather(node_p, src)
    cnts = _scatter_ones(dst, ones_rows, zeros_rows)

    ef_p = jnp.pad(edge_feat, ((0, epad), (0, 0)))
    wt = W_edge.T
    bt = b_edge.reshape(IN_DIM * OUT_DIM, 1)
    msgst4 = _dense(ef_p, xjt4, wt, bt)

    sums = _scatter(msgst4, dst, zeros_rows)

    eye = jnp.eye(IN_DIM, dtype=jnp.float32)
    wbig = jnp.kron(eye, W_root)
    fold = jnp.kron(jnp.ones((IN_DIM, 1), jnp.float32), eye)
    unfold = fold.T
    bbig = jnp.tile(b_root, IN_DIM).reshape(1, IN_DIM * OUT_DIM)
    gbig = jnp.tile(bn_gamma, IN_DIM).reshape(1, IN_DIM * OUT_DIM)
    betab = jnp.tile(bn_beta, IN_DIM).reshape(1, IN_DIM * OUT_DIM)

    lanes = IN_DIM * OUT_DIM
    out_r = _finish(
        sums.reshape(NC, NR, lanes), cnts.reshape(NC, NR, lanes),
        node_p.reshape(NR, lanes), wbig, bbig, gbig, betab, fold, unfold)
    return out_r.reshape(N_PAD, OUT_DIM)[:N]


# speedup vs baseline: 1.5007x; 1.2222x over previous
import jax
import jax.numpy as jnp
from jax import lax
from jax.experimental import pallas as pl
from jax.experimental.pallas import tpu as pltpu
from jax.experimental.pallas import tpu_sc as plsc

N = 50000
E = 800000
IN_DIM = 16
OUT_DIM = 16
EDGE_DIM = 16

NC = 2
NS = 16
NW = NC * NS

ICHUNK = 128
IROWS = 6272
E_PAD = IROWS * ICHUNK
NSUB = 3136
N_PAD = NS * NSUB
NR = N_PAD // 16
NR_REAL = N // 16

WROWS = IROWS // NW
BROWS = 7
NBLK = WROWS // BROWS

BE = 4096

_sc_mesh = plsc.VectorSubcoreMesh(core_axis_name="c", subcore_axis_name="s")
_sc_params = pltpu.CompilerParams(use_tc_tiling_on_sc=False,
                                  needs_layout_passes=False)


def _gather_body(node_hbm, src_hbm, xjt_hbm,
                 idx_v, rows_v, colall_v, sem):
    cid = lax.axis_index("c")
    sid = lax.axis_index("s")
    wid = sid * NC + cid
    base = wid * WROWS

    lane_b = lax.iota(jnp.int32, 16) * BROWS

    def blk(j, _):
        row0 = base + j * BROWS
        pltpu.sync_copy(src_hbm.at[pl.ds(row0, BROWS)], idx_v)
        copies = [
            pltpu.async_copy(node_hbm.at[idx_v.at[jj]],
                             rows_v.at[pl.ds(jj * ICHUNK, ICHUNK)], sem)
            for jj in range(BROWS)
        ]
        for cp in copies:
            cp.wait()
        for e in range(BROWS * ICHUNK):
            vals = rows_v[e, :]
            plsc.store_scatter(
                colall_v,
                [lane_b + (e // 128), jnp.full((16,), e % 128, jnp.int32)],
                vals)
        wcopies = [
            pltpu.async_copy(colall_v.at[pl.ds((tr * 8 + r) * BROWS, BROWS),
                                         pl.ds(0, 128)],
                             xjt_hbm.at[tr, pl.ds(row0, BROWS), r], sem)
            for tr in range(2) for r in range(8)
        ]
        for cp in wcopies:
            cp.wait()
        return _

    lax.fori_loop(0, NBLK, blk, None)


_gather = pl.kernel(
    _gather_body,
    out_type=jax.ShapeDtypeStruct((2, IROWS, 8, 128), jnp.float32),
    mesh=_sc_mesh,
    compiler_params=_sc_params,
    scratch_types=[
        pltpu.VMEM((BROWS, ICHUNK), jnp.int32),
        pltpu.VMEM((BROWS * ICHUNK, IN_DIM), jnp.float32),
        pltpu.VMEM((16 * BROWS, 129), jnp.float32),
        pltpu.SemaphoreType.DMA,
    ],
)


def _scatter_ones_body(dst_hbm, ones_hbm, zeros_hbm, cnts_hbm,
                       idx_v, ones_v, node_v, acc):
    cid = lax.axis_index("c")
    sid = lax.axis_index("s")
    wid = sid * NC + cid
    base = wid * WROWS
    nrow0 = sid * NSUB

    pltpu.sync_copy(zeros_hbm, node_v)
    pltpu.sync_copy(node_v, acc.at[pl.ds(nrow0, NSUB)])
    pltpu.sync_copy(ones_hbm, ones_v)
    plsc.subcore_barrier()

    def blk(j, _):
        row0 = base + j * BROWS
        pltpu.sync_copy(dst_hbm.at[pl.ds(row0, BROWS)], idx_v)
        for jj in range(BROWS):
            pltpu.sync_copy(ones_v, acc.at[idx_v.at[jj]], add=True)
        return _

    lax.fori_loop(0, NBLK, blk, None)
    plsc.subcore_barrier()

    pltpu.sync_copy(acc.at[pl.ds(nrow0, NSUB)], node_v)
    pltpu.sync_copy(node_v, cnts_hbm.at[cid, pl.ds(nrow0, NSUB)])


_scatter_ones = pl.kernel(
    _scatter_ones_body,
    out_type=jax.ShapeDtypeStruct((NC, N_PAD, OUT_DIM), jnp.float32),
    mesh=_sc_mesh,
    compiler_params=_sc_params,
    scratch_types=[
        pltpu.VMEM((BROWS, ICHUNK), jnp.int32),
        pltpu.VMEM((ICHUNK, OUT_DIM), jnp.float32),
        pltpu.VMEM((NSUB, OUT_DIM), jnp.float32),
        pltpu.VMEM_SHARED((N_PAD, OUT_DIM), jnp.float32),
    ],
)


def _scatter_body(msgs_hbm, dst_hbm, zeros_hbm, sums_hbm,
                  idx_v, mall_v, msg_v, node_v, acc, sem):
    cid = lax.axis_index("c")
    sid = lax.axis_index("s")
    wid = sid * NC + cid
    base = wid * WROWS
    nrow0 = sid * NSUB

    pltpu.sync_copy(zeros_hbm, node_v)
    pltpu.sync_copy(node_v, acc.at[pl.ds(nrow0, NSUB)])
    plsc.subcore_barrier()

    lane = lax.iota(jnp.int32, 16)
    feat_row = lane * BROWS

    def blk(j, _):
        row0 = base + j * BROWS
        pltpu.sync_copy(dst_hbm.at[pl.ds(row0, BROWS)], idx_v)
        rcopies = [
            pltpu.async_copy(msgs_hbm.at[tr, pl.ds(row0, BROWS), r],
                             mall_v.at[pl.ds((tr * 8 + r) * BROWS, BROWS),
                                       pl.ds(0, 128)], sem)
            for tr in range(2) for r in range(8)
        ]
        for cp in rcopies:
            cp.wait()
        for e in range(BROWS * ICHUNK):
            vals = plsc.load_gather(
                mall_v, [feat_row + (e // 128),
                         jnp.full((16,), e % 128, jnp.int32)])
            msg_v[e, :] = vals
        scopies = [
            pltpu.async_copy(msg_v.at[pl.ds(jj * ICHUNK, ICHUNK)],
                             acc.at[idx_v.at[jj]], sem, add=True)
            for jj in range(BROWS)
        ]
        for cp in scopies:
            cp.wait()
        return _

    lax.fori_loop(0, NBLK, blk, None)
    plsc.subcore_barrier()

    pltpu.sync_copy(acc.at[pl.ds(nrow0, NSUB)], node_v)
    pltpu.sync_copy(node_v, sums_hbm.at[cid, pl.ds(nrow0, NSUB)])


_scatter = pl.kernel(
    _scatter_body,
    out_type=jax.ShapeDtypeStruct((NC, N_PAD, OUT_DIM), jnp.float32),
    mesh=_sc_mesh,
    compiler_params=_sc_params,
    scratch_types=[
        pltpu.VMEM((BROWS, ICHUNK), jnp.int32),
        pltpu.VMEM((16 * BROWS, 129), jnp.float32),
        pltpu.VMEM((BROWS * ICHUNK, OUT_DIM), jnp.float32),
        pltpu.VMEM((NSUB, OUT_DIM), jnp.float32),
        pltpu.VMEM_SHARED((N_PAD, OUT_DIM), jnp.float32),
        pltpu.SemaphoreType.DMA,
    ],
)


_CONTRACT_LAST = (((1,), (1,)), ((), ()))
NBCH = BE // 128


def _dense_body(ef_ref, xjt_ref, wt_ref, bt_ref, out_ref):
    t = jnp.tanh(
        lax.dot_general(wt_ref[...], ef_ref[...], _CONTRACT_LAST,
                        preferred_element_type=jnp.float32) + bt_ref[...])
    for l in range(NBCH):
        tl = t[:, l * 128:(l + 1) * 128]
        acc = None
        for i in range(IN_DIM):
            xr = xjt_ref[i // 8, l, i % 8]
            xb = jnp.broadcast_to(xr[None, :], (OUT_DIM, 128))
            term = xb * tl[i * OUT_DIM:(i + 1) * OUT_DIM, :]
            acc = term if acc is None else acc + term
        out_ref[:, l, :, :] = acc.reshape(2, 8, 128)


def _dense(ef, xjt4, wt, bt):
    grid = (E_PAD // BE,)
    return pl.pallas_call(
        _dense_body,
        grid=grid,
        in_specs=[
            pl.BlockSpec((BE, EDGE_DIM), lambda i: (i, 0)),
            pl.BlockSpec((2, NBCH, 8, 128), lambda i: (0, i, 0, 0)),
            pl.BlockSpec((IN_DIM * OUT_DIM, EDGE_DIM), lambda i: (0, 0)),
            pl.BlockSpec((IN_DIM * OUT_DIM, 1), lambda i: (0, 0)),
        ],
        out_specs=pl.BlockSpec((2, NBCH, 8, 128), lambda i: (0, i, 0, 0)),
        out_shape=jax.ShapeDtypeStruct((2, IROWS, 8, 128), jnp.float32),
    )(ef, xjt4, wt, bt)


def _finish_body(sums_ref, cnts_ref, node_ref, wbig_ref, bbig_ref,
                 gbig_ref, betab_ref, fold_ref, unfold_ref, out_ref):
    s = sums_ref[0] + sums_ref[1]
    c = cnts_ref[0] + cnts_ref[1]
    aggr = s / jnp.maximum(c, 1.0)
    root = jnp.dot(node_ref[...], wbig_ref[...],
                   preferred_element_type=jnp.float32,
                   precision=lax.Precision.HIGHEST)
    pre = aggr + root + bbig_ref[...]
    rid = lax.broadcasted_iota(jnp.int32, (NR, IN_DIM * OUT_DIM), 0)
    pre = jnp.where(rid < NR_REAL, pre, 0.0)
    colsum = jnp.sum(pre, axis=0, keepdims=True)
    colsq = jnp.sum(pre * pre, axis=0, keepdims=True)
    tot = jnp.dot(colsum, fold_ref[...], preferred_element_type=jnp.float32,
                  precision=lax.Precision.HIGHEST)
    totsq = jnp.dot(colsq, fold_ref[...], preferred_element_type=jnp.float32,
                    precision=lax.Precision.HIGHEST)
    mean16 = tot / float(N)
    var16 = totsq / float(N) - mean16 * mean16
    mean_b = jnp.dot(mean16, unfold_ref[...],
                     preferred_element_type=jnp.float32,
                     precision=lax.Precision.HIGHEST)
    var_b = jnp.dot(var16, unfold_ref[...],
                    preferred_element_type=jnp.float32,
                    precision=lax.Precision.HIGHEST)
    y = (pre - mean_b) * lax.rsqrt(var_b + 1e-5) * gbig_ref[...] \
        + betab_ref[...]
    out_ref[...] = jnp.where(y >= 0.0, y, 0.01 * y)


def _finish(sums_r, cnts_r, node_r, wbig, bbig, gbig, betab, fold, unfold):
    return pl.pallas_call(
        _finish_body,
        out_shape=jax.ShapeDtypeStruct((NR, IN_DIM * OUT_DIM), jnp.float32),
    )(sums_r, cnts_r, node_r, wbig, bbig, gbig, betab, fold, unfold)


def kernel(node_feat, edge_feat, edge_index, batch_index,
           W_edge, b_edge, W_root, b_root, bn_gamma, bn_beta):
    del batch_index
    epad = E_PAD - E
    src = jnp.concatenate(
        [edge_index[0], jnp.zeros((epad,), edge_index.dtype)]
    ).astype(jnp.int32).reshape(IROWS, ICHUNK)
    dst = jnp.concatenate(
        [edge_index[1], jnp.full((epad,), N, edge_index.dtype)]
    ).astype(jnp.int32).reshape(IROWS, ICHUNK)

    node_p = jnp.pad(node_feat, ((0, N_PAD - N), (0, 0)))
    ones_rows = jnp.ones((ICHUNK, OUT_DIM), jnp.float32)
    zeros_rows = jnp.zeros((NSUB, OUT_DIM), jnp.float32)

    xjt4 = _gather(node_p, src)
    cnts = _scatter_ones(dst, ones_rows, zeros_rows)

    ef_p = jnp.pad(edge_feat, ((0, epad), (0, 0)))
    wt = W_edge.T
    bt = b_edge.reshape(IN_DIM * OUT_DIM, 1)
    msgst4 = _dense(ef_p, xjt4, wt, bt)

    sums = _scatter(msgst4, dst, zeros_rows)

    eye = jnp.eye(IN_DIM, dtype=jnp.float32)
    wbig = jnp.kron(eye, W_root)
    fold = jnp.kron(jnp.ones((IN_DIM, 1), jnp.float32), eye)
    unfold = fold.T
    bbig = jnp.tile(b_root, IN_DIM).reshape(1, IN_DIM * OUT_DIM)
    gbig = jnp.tile(bn_gamma, IN_DIM).reshape(1, IN_DIM * OUT_DIM)
    betab = jnp.tile(bn_beta, IN_DIM).reshape(1, IN_DIM * OUT_DIM)

    lanes = IN_DIM * OUT_DIM
    out_r = _finish(
        sums.reshape(NC, NR, lanes), cnts.reshape(NC, NR, lanes),
        node_p.reshape(NR, lanes), wbig, bbig, gbig, betab, fold, unfold)
    return out_r.reshape(N_PAD, OUT_DIM)[:N]
